# baseline (speedup 1.0000x reference)
import functools

import jax
import jax.numpy as jnp
from jax import lax
from jax.experimental import pallas as pl
from jax.experimental.pallas import tpu as pltpu
from jax.experimental.pallas import tpu_sc as plsc

B = 4096
L = 200
V = 100000
D = 128

NC = 2
NS = 16
LANES = 16
NW = NC * NS
BPW = B // NW
NCHUNK = 2
CH = L // NCHUNK
NBUF = 2
DW = D // 2
NVREG = D // 32


def _bow_body(idx_hbm, table_hbm, out_hbm, idx_v, buf_v, out_v, sem0, sem1):
    wid = lax.axis_index("s") * NC + lax.axis_index("c")
    sems = (sem0, sem1)
    inv = jnp.full((LANES,), 1.0 / L, dtype=jnp.float32)

    pltpu.sync_copy(idx_hbm.at[wid], idx_v)

    def start_gather(slot, bag):
        for c in range(NCHUNK):
            pltpu.make_async_copy(
                table_hbm.at[idx_v.at[bag * NCHUNK + c]],
                buf_v.at[slot, c],
                sems[slot],
            ).start()

    def drain(slot):
        for c in range(NCHUNK):
            pltpu.make_async_copy(
                table_hbm.at[idx_v.at[0]],
                buf_v.at[slot, c],
                sems[slot],
            ).wait()

    def consume(slot, bag):
        UNROLL = 2

        def row_add(i, accs):
            l = i * UNROLL
            out = []
            for k in range(NVREG):
                lo, hi = accs[2 * k], accs[2 * k + 1]
                los, his = [], []
                for u in range(UNROLL):
                    for c in range(NCHUNK):
                        x = buf_v[slot, c, l + u, pl.ds(k * LANES, LANES)]
                        a, b = plsc.unpack(
                            plsc.bitcast(x, jnp.bfloat16),
                            format=plsc.PackFormat.INTERLEAVED)
                        los.append(a)
                        his.append(b)
                while len(los) > 1:
                    los = [los[j] + los[j + 1] for j in range(0, len(los), 2)]
                    his = [his[j] + his[j + 1] for j in range(0, len(his), 2)]
                out.append(lo + los[0])
                out.append(hi + his[0])
            return tuple(out)

        accs = tuple(jnp.zeros((LANES,), jnp.float32) for _ in range(2 * NVREG))
        accs = lax.fori_loop(0, CH // UNROLL, row_add, accs)
        for k in range(NVREG):
            out_v[bag, pl.ds(2 * k * LANES, LANES)] = accs[2 * k] * inv
            out_v[bag, pl.ds((2 * k + 1) * LANES, LANES)] = accs[2 * k + 1] * inv

    for s in range(NBUF):
        start_gather(s, s)

    def step(i, _):
        for s in range(NBUF):
            bag = i * NBUF + s
            drain(s)
            consume(s, bag)
            start_gather(s, bag + NBUF)
        return 0

    lax.fori_loop(0, BPW // NBUF - 1, step, 0)

    for s in range(NBUF):
        bag = BPW - NBUF + s
        drain(s)
        consume(s, bag)

    pltpu.sync_copy(out_v, out_hbm.at[pl.ds(wid * BPW, BPW)])


@jax.jit
def _bow(idx_r, table_p):
    mesh = plsc.VectorSubcoreMesh(core_axis_name="c", subcore_axis_name="s")
    return pl.kernel(
        _bow_body,
        mesh=mesh,
        compiler_params=pltpu.CompilerParams(
            needs_layout_passes=False, use_tc_tiling_on_sc=False),
        out_type=jax.ShapeDtypeStruct((B, D), jnp.float32),
        scratch_types=[
            pltpu.VMEM((BPW * NCHUNK, CH), jnp.int32),
            pltpu.VMEM((NBUF, NCHUNK, CH, DW), jnp.int32),
            pltpu.VMEM((BPW, D), jnp.float32),
            pltpu.SemaphoreType.DMA,
            pltpu.SemaphoreType.DMA,
        ],
    )(idx_r, table_p)


def kernel(indices, table):
    idx_r = indices.reshape(NW, BPW * NCHUNK, CH)
    table_p = table.reshape(V, D // 32, 2, 16).swapaxes(2, 3).reshape(V, D)
    table_p = table_p.astype(jnp.bfloat16).reshape(V, DW, 2)
    table_p = lax.bitcast_convert_type(table_p, jnp.int32)
    return _bow(idx_r, table_p)
